# Initial kernel scaffold; baseline (speedup 1.0000x reference)
import functools

import jax
import jax.numpy as jnp
from jax import lax
from jax.experimental import pallas as pl
from jax.experimental.pallas import tpu as pltpu
from jax.experimental.pallas import tpu_sc as plsc

N = 10000
NP = 10240
E = 160000
EP = 163840
CH = 128
KCH = EP // (32 * CH)
RSUB = NP // 16

_MESH = plsc.VectorSubcoreMesh(core_axis_name="c", subcore_axis_name="s")


def _zero_fill(buf, nrows, ncols):
    @pl.loop(0, nrows)
    def _(i):
        for j in range(ncols // 16):
            buf[i, pl.ds(j * 16, 16)] = jnp.zeros((16,), jnp.float32)


@functools.partial(
    pl.kernel,
    out_type=jax.ShapeDtypeStruct((2, NP, 16), jnp.float32),
    mesh=_MESH,
    scratch_types=[
        pltpu.VMEM((KCH, CH), jnp.int32),
        pltpu.VMEM((CH, 16), jnp.float32),
        pltpu.VMEM((64, 16), jnp.float32),
        pltpu.VMEM_SHARED((NP, 16), jnp.float32),
    ],
)
def _sc_degree(dstw_hbm, out_hbm, dst_v, ones_v, zero_v, acc_sh):
    c = lax.axis_index("c")
    s = lax.axis_index("s")
    wid = c * 16 + s
    pltpu.sync_copy(dstw_hbm.at[wid], dst_v)

    @pl.loop(0, CH)
    def _(i):
        ones_v[i, :] = jnp.full((16,), 1.0, jnp.float32)

    _zero_fill(zero_v, 64, 16)

    @pl.loop(0, RSUB // 64)
    def _(j):
        pltpu.sync_copy(zero_v, acc_sh.at[pl.ds(s * RSUB + j * 64, 64)])

    plsc.subcore_barrier()

    @pl.loop(0, KCH)
    def _(k):
        pltpu.sync_copy(ones_v, acc_sh.at[dst_v.at[k]], add=True)

    plsc.subcore_barrier()
    pltpu.sync_copy(acc_sh.at[pl.ds(s * RSUB, RSUB)],
                    out_hbm.at[c, pl.ds(s * RSUB, RSUB)])


def _make_sc_agg(F):
    @functools.partial(
        pl.kernel,
        out_type=jax.ShapeDtypeStruct((2, NP, F), jnp.float32),
        mesh=_MESH,
        scratch_types=[
            pltpu.VMEM((KCH, CH), jnp.int32),
            pltpu.VMEM((KCH, CH), jnp.int32),
            pltpu.VMEM((CH, F), jnp.float32),
            pltpu.VMEM((64, F), jnp.float32),
            pltpu.VMEM_SHARED((NP, F), jnp.float32),
        ],
    )
    def _sc_agg(xs_hbm, srcw_hbm, dstw_hbm, out_hbm,
                src_v, dst_v, rows_v, zero_v, acc_sh):
        c = lax.axis_index("c")
        s = lax.axis_index("s")
        wid = c * 16 + s
        pltpu.sync_copy(srcw_hbm.at[wid], src_v)
        pltpu.sync_copy(dstw_hbm.at[wid], dst_v)

        _zero_fill(zero_v, 64, F)

        @pl.loop(0, RSUB // 64)
        def _(j):
            pltpu.sync_copy(zero_v, acc_sh.at[pl.ds(s * RSUB + j * 64, 64)])

        plsc.subcore_barrier()

        @pl.loop(0, KCH)
        def _(k):
            pltpu.sync_copy(xs_hbm.at[src_v.at[k]], rows_v)
            pltpu.sync_copy(rows_v, acc_sh.at[dst_v.at[k]], add=True)

        plsc.subcore_barrier()
        pltpu.sync_copy(acc_sh.at[pl.ds(s * RSUB, RSUB)],
                        out_hbm.at[c, pl.ds(s * RSUB, RSUB)])

    return _sc_agg


_sc_agg32 = _make_sc_agg(32)
_sc_agg64 = _make_sc_agg(64)


_BR = 1024


def _mm1_body(x_ref, w_ref, o_ref):
    o_ref[...] = jnp.dot(x_ref[...], w_ref[...],
                         preferred_element_type=jnp.float32)


def _tc_mm1(x, w):
    return pl.pallas_call(
        _mm1_body,
        grid=(NP // _BR,),
        in_specs=[
            pl.BlockSpec((_BR, 256), lambda i: (i, 0)),
            pl.BlockSpec((256, 32), lambda i: (0, 0)),
        ],
        out_specs=pl.BlockSpec((_BR, 32), lambda i: (i, 0)),
        out_shape=jax.ShapeDtypeStruct((NP, 32), jnp.float32),
    )(x, w)


def _dis_from(deg_ref):
    d = deg_ref[0, :, 0] + deg_ref[1, :, 0] + 1.0
    return lax.rsqrt(d)


def _scale_body(xw_ref, deg_ref, o_ref):
    dis = _dis_from(deg_ref)
    o_ref[...] = xw_ref[...] * dis[:, None]


def _tc_scale(xw, deg):
    return pl.pallas_call(
        _scale_body,
        grid=(NP // _BR,),
        in_specs=[
            pl.BlockSpec((_BR, 32), lambda i: (i, 0)),
            pl.BlockSpec((2, _BR, 16), lambda i: (0, i, 0)),
        ],
        out_specs=pl.BlockSpec((_BR, 32), lambda i: (i, 0)),
        out_shape=jax.ShapeDtypeStruct((NP, 32), jnp.float32),
    )(xw, deg)


def _mid_body(p_ref, xw1_ref, deg_ref, b1_ref, w2_ref, xw2_ref, xs2_ref):
    dis = _dis_from(deg_ref)
    xw1 = xw1_ref[...]
    h1 = dis[:, None] * (p_ref[0] + p_ref[1]) + (dis * dis)[:, None] * xw1
    h1 = jnp.maximum(h1 + b1_ref[0][None, :], 0.0)
    xw2 = jnp.dot(h1, w2_ref[...], preferred_element_type=jnp.float32)
    xw2_ref[...] = xw2
    xs2_ref[...] = xw2 * dis[:, None]


def _tc_mid(p, xw1, deg, b1, w2):
    return pl.pallas_call(
        _mid_body,
        grid=(NP // _BR,),
        in_specs=[
            pl.BlockSpec((2, _BR, 32), lambda i: (0, i, 0)),
            pl.BlockSpec((_BR, 32), lambda i: (i, 0)),
            pl.BlockSpec((2, _BR, 16), lambda i: (0, i, 0)),
            pl.BlockSpec((1, 32), lambda i: (0, 0)),
            pl.BlockSpec((32, 64), lambda i: (0, 0)),
        ],
        out_specs=[
            pl.BlockSpec((_BR, 64), lambda i: (i, 0)),
            pl.BlockSpec((_BR, 64), lambda i: (i, 0)),
        ],
        out_shape=[
            jax.ShapeDtypeStruct((NP, 64), jnp.float32),
            jax.ShapeDtypeStruct((NP, 64), jnp.float32),
        ],
    )(p, xw1, deg, b1, w2)


def _out_body(q_ref, xw2_ref, deg_ref, b2_ref, o_ref):
    dis = _dis_from(deg_ref)
    logits = dis[:, None] * (q_ref[0] + q_ref[1]) \
        + (dis * dis)[:, None] * xw2_ref[...]
    logits = jnp.maximum(logits + b2_ref[0][None, :], 0.0)
    m = jnp.max(logits, axis=1, keepdims=True)
    e = jnp.exp(logits - m)
    o_ref[...] = e / jnp.sum(e, axis=1, keepdims=True)


def _tc_out(q, xw2, deg, b2):
    return pl.pallas_call(
        _out_body,
        grid=(NP // _BR,),
        in_specs=[
            pl.BlockSpec((2, _BR, 64), lambda i: (0, i, 0)),
            pl.BlockSpec((_BR, 64), lambda i: (i, 0)),
            pl.BlockSpec((2, _BR, 16), lambda i: (0, i, 0)),
            pl.BlockSpec((1, 64), lambda i: (0, 0)),
        ],
        out_specs=pl.BlockSpec((_BR, 64), lambda i: (i, 0)),
        out_shape=jax.ShapeDtypeStruct((NP, 64), jnp.float32),
    )(q, xw2, deg, b2)


@jax.jit
def kernel(x, edge_index, W1, b1, W2, b2):
    ei = edge_index.astype(jnp.int32)
    src = jnp.concatenate([ei[0], jnp.zeros((EP - E,), jnp.int32)])
    dst = jnp.concatenate([ei[1], jnp.full((EP - E,), NP - 1, jnp.int32)])
    srcw = src.reshape(32, KCH, CH)
    dstw = dst.reshape(32, KCH, CH)
    xp = jnp.concatenate([x, jnp.zeros((NP - N, x.shape[1]), x.dtype)])

    deg = _sc_degree(dstw)
    xw1 = _tc_mm1(xp, W1)
    xs1 = _tc_scale(xw1, deg)
    p = _sc_agg32(xs1, srcw, dstw)
    xw2, xs2 = _tc_mid(p, xw1, deg, b1.reshape(1, 32), W2)
    q = _sc_agg64(xs2, srcw, dstw)
    probs = _tc_out(q, xw2, deg, b2.reshape(1, 64))
    return probs[:N]


# speedup vs baseline: 5.9771x; 5.9771x over previous
import dataclasses
import functools

import jax
import jax.numpy as jnp
from jax import lax
from jax.experimental import pallas as pl
from jax.experimental.pallas import tpu as pltpu
from jax.experimental.pallas import tpu_sc as plsc

N = 10000
NP = 10240
E = 160000
EP = 163840
CH = 128
NCHUNK = EP // CH
IG = 8
RTILE = NP // 32

_MESH = plsc.VectorSubcoreMesh(core_axis_name="c", subcore_axis_name="s")

_CP = pltpu.CompilerParams()
if "needs_layout_passes" in pltpu.CompilerParams.__dataclass_fields__:
    _CP = dataclasses.replace(_CP, needs_layout_passes=False)


@functools.partial(
    pl.kernel,
    out_type=jax.ShapeDtypeStruct((32 * NP,), jnp.float32),
    mesh=_MESH,
    scratch_types=[
        pltpu.VMEM((IG, CH), jnp.int32),
        pltpu.VMEM((NP,), jnp.float32),
    ],
    compiler_params=_CP,
)
def _sc_degree_part(dstw_hbm, part_hbm, idx_v, hist_v):
    c = lax.axis_index("c")
    s = lax.axis_index("s")
    wid = c * 16 + s

    @pl.loop(0, NP // 16)
    def _(r):
        hist_v[pl.ds(r * 16, 16)] = jnp.zeros((16,), jnp.float32)

    @pl.loop(0, (NCHUNK // 32) // IG)
    def _(g):
        pltpu.sync_copy(dstw_hbm.at[pl.ds(wid * (NCHUNK // 32) + g * IG, IG)],
                        idx_v)
        for i in range(IG):
            for v in range(CH // 16):
                d16 = idx_v[i, pl.ds(v * 16, 16)]
                plsc.addupdate_scatter(hist_v, [d16],
                                       jnp.full((16,), 1.0, jnp.float32))

    pltpu.sync_copy(hist_v, part_hbm.at[pl.ds(wid * NP, NP)])


@functools.partial(
    pl.kernel,
    out_type=jax.ShapeDtypeStruct((NP // 2, 128), jnp.float32),
    mesh=_MESH,
    scratch_types=[
        pltpu.VMEM((RTILE,), jnp.float32),
        pltpu.VMEM((RTILE,), jnp.float32),
        pltpu.VMEM((16, 128), jnp.float32),
    ],
    compiler_params=_CP,
)
def _sc_degree_reduce(part_hbm, out_hbm, hbuf_v, red_v, outb_v):
    c = lax.axis_index("c")
    s = lax.axis_index("s")
    wid = c * 16 + s

    @pl.loop(0, RTILE // 16)
    def _(r):
        red_v[pl.ds(r * 16, 16)] = jnp.zeros((16,), jnp.float32)

    for w in range(32):
        pltpu.sync_copy(part_hbm.at[pl.ds(w * NP + wid * RTILE, RTILE)],
                        hbuf_v)

        @pl.loop(0, RTILE // 16)
        def _(r):
            red_v[pl.ds(r * 16, 16)] = (red_v[pl.ds(r * 16, 16)]
                                        + hbuf_v[pl.ds(r * 16, 16)])

    @pl.loop(0, RTILE // 32)
    def _(blk):
        @pl.loop(0, 16)
        def _(a):
            for b in range(2):
                ii = jnp.full((16,), blk * 32 + a * 2 + b, jnp.int32)
                vv = plsc.load_gather(red_v, [ii])
                for t in range(4):
                    outb_v[a, pl.ds(b * 64 + t * 16, 16)] = vv

        pltpu.sync_copy(outb_v,
                        out_hbm.at[pl.ds(wid * (RTILE // 2) + blk * 16, 16)])


def _sc_degree(dstw):
    return _sc_degree_reduce(_sc_degree_part(dstw))


def _make_sc_agg(F):
    CPT = F // 32

    @functools.partial(
        pl.kernel,
        out_type=jax.ShapeDtypeStruct((F * NP,), jnp.float32),
        mesh=_MESH,
        scratch_types=[
            pltpu.VMEM((IG, CH), jnp.int32),
            pltpu.VMEM((IG, CH), jnp.int32),
            pltpu.VMEM((CPT * NP,), jnp.float32),
            pltpu.VMEM((CPT * NP,), jnp.float32),
        ],
        compiler_params=_CP,
    )
    def _sc_agg(xst_hbm, srcw_hbm, dstw_hbm, out_hbm,
                sidx_v, didx_v, xs_v, acc_v):
        c = lax.axis_index("c")
        s = lax.axis_index("s")
        wid = c * 16 + s

        for t in range(CPT):
            pltpu.sync_copy(xst_hbm.at[pl.ds((wid * CPT + t) * NP, NP)],
                            xs_v.at[pl.ds(t * NP, NP)])

        @pl.loop(0, (CPT * NP) // 16)
        def _(r):
            acc_v[pl.ds(r * 16, 16)] = jnp.zeros((16,), jnp.float32)

        @pl.loop(0, NCHUNK // IG)
        def _(g):
            pltpu.sync_copy(srcw_hbm.at[pl.ds(g * IG, IG)], sidx_v)
            pltpu.sync_copy(dstw_hbm.at[pl.ds(g * IG, IG)], didx_v)
            for i in range(IG):
                for v in range(CH // 16):
                    s16 = sidx_v[i, pl.ds(v * 16, 16)]
                    d16 = didx_v[i, pl.ds(v * 16, 16)]
                    for t in range(CPT):
                        off = t * NP
                        vals = plsc.load_gather(xs_v, [s16 + off])
                        plsc.addupdate_scatter(acc_v, [d16 + off], vals)

        for t in range(CPT):
            pltpu.sync_copy(acc_v.at[pl.ds(t * NP, NP)],
                            out_hbm.at[pl.ds((wid * CPT + t) * NP, NP)])

    return _sc_agg


_sc_agg1 = _make_sc_agg(32)
_sc_agg2 = _make_sc_agg(64)


_BR = 1024


def _dis_from(deg_ref):
    return lax.rsqrt(deg_ref[...] + 1.0)


def _mm1_body(x_ref, w_ref, o_ref):
    o_ref[...] = jnp.dot(x_ref[...], w_ref[...],
                         preferred_element_type=jnp.float32)


def _tc_mm1(x, w):
    return pl.pallas_call(
        _mm1_body,
        grid=(NP // _BR,),
        in_specs=[
            pl.BlockSpec((_BR, 256), lambda i: (i, 0)),
            pl.BlockSpec((256, 32), lambda i: (0, 0)),
        ],
        out_specs=pl.BlockSpec((_BR, 32), lambda i: (i, 0)),
        out_shape=jax.ShapeDtypeStruct((NP, 32), jnp.float32),
    )(x, w)


def _scale_body(xw_ref, deg_ref, o_ref):
    dis = _dis_from(deg_ref)
    o_ref[...] = xw_ref[...] * dis[:, :32]


def _tc_scale(xw, deg):
    return pl.pallas_call(
        _scale_body,
        grid=(NP // _BR,),
        in_specs=[
            pl.BlockSpec((_BR, 32), lambda i: (i, 0)),
            pl.BlockSpec((_BR, 64), lambda i: (i, 0)),
        ],
        out_specs=pl.BlockSpec((_BR, 32), lambda i: (i, 0)),
        out_shape=jax.ShapeDtypeStruct((NP, 32), jnp.float32),
    )(xw, deg)


def _mid_body(p_ref, xw1_ref, deg_ref, b1_ref, w2_ref, xw2_ref, xs2_ref):
    dis = _dis_from(deg_ref)
    d32 = dis[:, :32]
    h1 = d32 * p_ref[...] + d32 * d32 * xw1_ref[...]
    h1 = jnp.maximum(h1 + b1_ref[0][None, :], 0.0)
    xw2 = jnp.dot(h1, w2_ref[...], preferred_element_type=jnp.float32)
    xw2_ref[...] = xw2
    xs2_ref[...] = xw2 * dis


def _tc_mid(p, xw1, deg, b1, w2):
    return pl.pallas_call(
        _mid_body,
        grid=(NP // _BR,),
        in_specs=[
            pl.BlockSpec((_BR, 32), lambda i: (i, 0)),
            pl.BlockSpec((_BR, 32), lambda i: (i, 0)),
            pl.BlockSpec((_BR, 64), lambda i: (i, 0)),
            pl.BlockSpec((1, 32), lambda i: (0, 0)),
            pl.BlockSpec((32, 64), lambda i: (0, 0)),
        ],
        out_specs=[
            pl.BlockSpec((_BR, 64), lambda i: (i, 0)),
            pl.BlockSpec((_BR, 64), lambda i: (i, 0)),
        ],
        out_shape=[
            jax.ShapeDtypeStruct((NP, 64), jnp.float32),
            jax.ShapeDtypeStruct((NP, 64), jnp.float32),
        ],
    )(p, xw1, deg, b1, w2)


def _out_body(q_ref, xw2_ref, deg_ref, b2_ref, o_ref):
    dis = _dis_from(deg_ref)
    logits = dis * q_ref[...] + dis * dis * xw2_ref[...]
    logits = jnp.maximum(logits + b2_ref[0][None, :], 0.0)
    m = jnp.max(logits, axis=1, keepdims=True)
    e = jnp.exp(logits - m)
    o_ref[...] = e / jnp.sum(e, axis=1, keepdims=True)


def _tc_out(q, xw2, deg, b2):
    return pl.pallas_call(
        _out_body,
        grid=(NP // _BR,),
        in_specs=[
            pl.BlockSpec((_BR, 64), lambda i: (i, 0)),
            pl.BlockSpec((_BR, 64), lambda i: (i, 0)),
            pl.BlockSpec((_BR, 64), lambda i: (i, 0)),
            pl.BlockSpec((1, 64), lambda i: (0, 0)),
        ],
        out_specs=pl.BlockSpec((_BR, 64), lambda i: (i, 0)),
        out_shape=jax.ShapeDtypeStruct((NP, 64), jnp.float32),
    )(q, xw2, deg, b2)


@jax.jit
def kernel(x, edge_index, W1, b1, W2, b2):
    ei = edge_index.astype(jnp.int32)
    src = jnp.concatenate([ei[0], jnp.zeros((EP - E,), jnp.int32)])
    dst = jnp.concatenate([ei[1], jnp.full((EP - E,), NP - 1, jnp.int32)])
    srcw = src.reshape(NCHUNK, CH)
    dstw = dst.reshape(NCHUNK, CH)
    xp = jnp.concatenate([x, jnp.zeros((NP - N, x.shape[1]), x.dtype)])

    deg = _sc_degree(dstw)
    degr = deg.reshape(NP, 64)
    xw1 = _tc_mm1(xp, W1)
    xs1 = _tc_scale(xw1, degr)
    p = _sc_agg1(xs1.T.reshape(-1), srcw, dstw)
    xw2, xs2 = _tc_mid(p.reshape(32, NP).T, xw1, degr,
                       b1.reshape(1, 32), W2)
    q = _sc_agg2(xs2.T.reshape(-1), srcw, dstw)
    probs = _tc_out(q.reshape(64, NP).T, xw2, degr, b2.reshape(1, 64))
    return probs[:N]


# speedup vs baseline: 8.3982x; 1.4051x over previous
import dataclasses
import functools

import jax
import jax.numpy as jnp
from jax import lax
from jax.experimental import pallas as pl
from jax.experimental.pallas import tpu as pltpu
from jax.experimental.pallas import tpu_sc as plsc

N = 10000
NP = 10240
E = 160000
EP = 163840
CH = 128
NCHUNK = EP // CH
IG = 8
IGA = 32
RTILE = NP // 32

_MESH = plsc.VectorSubcoreMesh(core_axis_name="c", subcore_axis_name="s")

_CP = pltpu.CompilerParams()
if "needs_layout_passes" in pltpu.CompilerParams.__dataclass_fields__:
    _CP = dataclasses.replace(_CP, needs_layout_passes=False)


@functools.partial(
    pl.kernel,
    out_type=jax.ShapeDtypeStruct((32 * NP,), jnp.float32),
    mesh=_MESH,
    scratch_types=[
        pltpu.VMEM((IG, CH), jnp.int32),
        pltpu.VMEM((NP,), jnp.float32),
    ],
    compiler_params=_CP,
)
def _sc_degree_part(dstw_hbm, part_hbm, idx_v, hist_v):
    c = lax.axis_index("c")
    s = lax.axis_index("s")
    wid = c * 16 + s

    @pl.loop(0, NP // 16)
    def _(r):
        hist_v[pl.ds(r * 16, 16)] = jnp.zeros((16,), jnp.float32)

    @pl.loop(0, (NCHUNK // 32) // IG)
    def _(g):
        pltpu.sync_copy(dstw_hbm.at[pl.ds(wid * (NCHUNK // 32) + g * IG, IG)],
                        idx_v)
        for i in range(IG):
            for v in range(CH // 16):
                d16 = idx_v[i, pl.ds(v * 16, 16)]
                plsc.addupdate_scatter(hist_v, [d16],
                                       jnp.full((16,), 1.0, jnp.float32))

    pltpu.sync_copy(hist_v, part_hbm.at[pl.ds(wid * NP, NP)])


@functools.partial(
    pl.kernel,
    out_type=jax.ShapeDtypeStruct((NP // 2, 128), jnp.float32),
    mesh=_MESH,
    scratch_types=[
        pltpu.VMEM((RTILE,), jnp.float32),
        pltpu.VMEM((RTILE,), jnp.float32),
        pltpu.VMEM((16, 128), jnp.float32),
    ],
    compiler_params=_CP,
)
def _sc_degree_reduce(part_hbm, out_hbm, hbuf_v, red_v, outb_v):
    c = lax.axis_index("c")
    s = lax.axis_index("s")
    wid = c * 16 + s

    @pl.loop(0, RTILE // 16)
    def _(r):
        red_v[pl.ds(r * 16, 16)] = jnp.zeros((16,), jnp.float32)

    for w in range(32):
        pltpu.sync_copy(part_hbm.at[pl.ds(w * NP + wid * RTILE, RTILE)],
                        hbuf_v)

        @pl.loop(0, RTILE // 16)
        def _(r):
            red_v[pl.ds(r * 16, 16)] = (red_v[pl.ds(r * 16, 16)]
                                        + hbuf_v[pl.ds(r * 16, 16)])

    @pl.loop(0, RTILE // 32)
    def _(blk):
        @pl.loop(0, 16)
        def _(a):
            for b in range(2):
                ii = jnp.full((16,), blk * 32 + a * 2 + b, jnp.int32)
                vv = plsc.load_gather(red_v, [ii])
                for t in range(4):
                    outb_v[a, pl.ds(b * 64 + t * 16, 16)] = vv

        pltpu.sync_copy(outb_v,
                        out_hbm.at[pl.ds(wid * (RTILE // 2) + blk * 16, 16)])


def _sc_degree(dstw):
    return _sc_degree_reduce(_sc_degree_part(dstw))


def _make_sc_agg(F):
    CPT = F // 32

    @functools.partial(
        pl.kernel,
        out_type=jax.ShapeDtypeStruct((F * NP,), jnp.float32),
        mesh=_MESH,
        scratch_types=[
            pltpu.VMEM((IGA, CH), jnp.int32),
            pltpu.VMEM((IGA, CH), jnp.int32),
            pltpu.VMEM((CPT * NP,), jnp.float32),
            pltpu.VMEM((CPT * NP,), jnp.float32),
        ],
        compiler_params=_CP,
    )
    def _sc_agg(xst_hbm, srcw_hbm, dstw_hbm, out_hbm,
                sidx_v, didx_v, xs_v, acc_v):
        c = lax.axis_index("c")
        s = lax.axis_index("s")
        wid = c * 16 + s

        for t in range(CPT):
            pltpu.sync_copy(xst_hbm.at[pl.ds((wid * CPT + t) * NP, NP)],
                            xs_v.at[pl.ds(t * NP, NP)])

        @pl.loop(0, (CPT * NP) // 16)
        def _(r):
            acc_v[pl.ds(r * 16, 16)] = jnp.zeros((16,), jnp.float32)

        @pl.loop(0, NCHUNK // IGA)
        def _(g):
            pltpu.sync_copy(srcw_hbm.at[pl.ds(g * IGA, IGA)], sidx_v)
            pltpu.sync_copy(dstw_hbm.at[pl.ds(g * IGA, IGA)], didx_v)
            for i in range(IGA):
                for v in range(CH // 16):
                    s16 = sidx_v[i, pl.ds(v * 16, 16)]
                    d16 = didx_v[i, pl.ds(v * 16, 16)]
                    for t in range(CPT):
                        off = t * NP
                        vals = plsc.load_gather(xs_v, [s16 + off])
                        plsc.addupdate_scatter(acc_v, [d16 + off], vals)

        for t in range(CPT):
            pltpu.sync_copy(acc_v.at[pl.ds(t * NP, NP)],
                            out_hbm.at[pl.ds((wid * CPT + t) * NP, NP)])

    return _sc_agg


_sc_agg1 = _make_sc_agg(32)
_sc_agg2 = _make_sc_agg(64)


_BR = 1024


def _dis_from(deg_ref):
    return lax.rsqrt(deg_ref[...] + 1.0)


def _mm1_body(x_ref, w_ref, o_ref):
    o_ref[...] = jnp.dot(x_ref[...], w_ref[...],
                         preferred_element_type=jnp.float32)


def _tc_mm1(x, w):
    return pl.pallas_call(
        _mm1_body,
        grid=(NP // _BR,),
        in_specs=[
            pl.BlockSpec((_BR, 256), lambda i: (i, 0)),
            pl.BlockSpec((256, 32), lambda i: (0, 0)),
        ],
        out_specs=pl.BlockSpec((_BR, 32), lambda i: (i, 0)),
        out_shape=jax.ShapeDtypeStruct((NP, 32), jnp.float32),
    )(x, w)


def _scale_body(xw_ref, deg_ref, o_ref):
    dis = _dis_from(deg_ref)
    o_ref[...] = xw_ref[...] * dis[:, :32]


def _tc_scale(xw, deg):
    return pl.pallas_call(
        _scale_body,
        grid=(NP // _BR,),
        in_specs=[
            pl.BlockSpec((_BR, 32), lambda i: (i, 0)),
            pl.BlockSpec((_BR, 64), lambda i: (i, 0)),
        ],
        out_specs=pl.BlockSpec((_BR, 32), lambda i: (i, 0)),
        out_shape=jax.ShapeDtypeStruct((NP, 32), jnp.float32),
    )(xw, deg)


def _mid_body(p_ref, xw1_ref, deg_ref, b1_ref, w2_ref, xw2_ref, xs2_ref):
    dis = _dis_from(deg_ref)
    d32 = dis[:, :32]
    h1 = d32 * p_ref[...] + d32 * d32 * xw1_ref[...]
    h1 = jnp.maximum(h1 + b1_ref[0][None, :], 0.0)
    xw2 = jnp.dot(h1, w2_ref[...], preferred_element_type=jnp.float32)
    xw2_ref[...] = xw2
    xs2_ref[...] = xw2 * dis


def _tc_mid(p, xw1, deg, b1, w2):
    return pl.pallas_call(
        _mid_body,
        grid=(NP // _BR,),
        in_specs=[
            pl.BlockSpec((_BR, 32), lambda i: (i, 0)),
            pl.BlockSpec((_BR, 32), lambda i: (i, 0)),
            pl.BlockSpec((_BR, 64), lambda i: (i, 0)),
            pl.BlockSpec((1, 32), lambda i: (0, 0)),
            pl.BlockSpec((32, 64), lambda i: (0, 0)),
        ],
        out_specs=[
            pl.BlockSpec((_BR, 64), lambda i: (i, 0)),
            pl.BlockSpec((_BR, 64), lambda i: (i, 0)),
        ],
        out_shape=[
            jax.ShapeDtypeStruct((NP, 64), jnp.float32),
            jax.ShapeDtypeStruct((NP, 64), jnp.float32),
        ],
    )(p, xw1, deg, b1, w2)


def _out_body(q_ref, xw2_ref, deg_ref, b2_ref, o_ref):
    dis = _dis_from(deg_ref)
    logits = dis * q_ref[...] + dis * dis * xw2_ref[...]
    logits = jnp.maximum(logits + b2_ref[0][None, :], 0.0)
    m = jnp.max(logits, axis=1, keepdims=True)
    e = jnp.exp(logits - m)
    o_ref[...] = e / jnp.sum(e, axis=1, keepdims=True)


def _tc_out(q, xw2, deg, b2):
    return pl.pallas_call(
        _out_body,
        grid=(NP // _BR,),
        in_specs=[
            pl.BlockSpec((_BR, 64), lambda i: (i, 0)),
            pl.BlockSpec((_BR, 64), lambda i: (i, 0)),
            pl.BlockSpec((_BR, 64), lambda i: (i, 0)),
            pl.BlockSpec((1, 64), lambda i: (0, 0)),
        ],
        out_specs=pl.BlockSpec((_BR, 64), lambda i: (i, 0)),
        out_shape=jax.ShapeDtypeStruct((NP, 64), jnp.float32),
    )(q, xw2, deg, b2)


@jax.jit
def kernel(x, edge_index, W1, b1, W2, b2):
    ei = edge_index.astype(jnp.int32)
    src = jnp.concatenate([ei[0], jnp.zeros((EP - E,), jnp.int32)])
    dst = jnp.concatenate([ei[1], jnp.full((EP - E,), NP - 1, jnp.int32)])
    srcw = src.reshape(NCHUNK, CH)
    dstw = dst.reshape(NCHUNK, CH)
    xp = jnp.concatenate([x, jnp.zeros((NP - N, x.shape[1]), x.dtype)])

    deg = _sc_degree(dstw)
    degr = deg.reshape(NP, 64)
    xw1 = _tc_mm1(xp, W1)
    xs1 = _tc_scale(xw1, degr)
    p = _sc_agg1(xs1.T.reshape(-1), srcw, dstw)
    xw2, xs2 = _tc_mid(p.reshape(32, NP).T, xw1, degr,
                       b1.reshape(1, 32), W2)
    q = _sc_agg2(xs2.T.reshape(-1), srcw, dstw)
    probs = _tc_out(q.reshape(64, NP).T, xw2, degr, b2.reshape(1, 64))
    return probs[:N]


# speedup vs baseline: 8.8402x; 1.0526x over previous
import dataclasses
import functools

import jax
import jax.numpy as jnp
from jax import lax
from jax.experimental import pallas as pl
from jax.experimental.pallas import tpu as pltpu
from jax.experimental.pallas import tpu_sc as plsc

N = 10000
NP = 10240
E = 160000
EP = 163840
CH = 128
NCHUNK = EP // CH
IG = 8
IGA = 64
RTILE = NP // 32

_MESH = plsc.VectorSubcoreMesh(core_axis_name="c", subcore_axis_name="s")

_CP = pltpu.CompilerParams()
if "needs_layout_passes" in pltpu.CompilerParams.__dataclass_fields__:
    _CP = dataclasses.replace(_CP, needs_layout_passes=False)


@functools.partial(
    pl.kernel,
    out_type=jax.ShapeDtypeStruct((32 * NP,), jnp.float32),
    mesh=_MESH,
    scratch_types=[
        pltpu.VMEM((IG, CH), jnp.int32),
        pltpu.VMEM((NP,), jnp.float32),
    ],
    compiler_params=_CP,
)
def _sc_degree_part(dstw_hbm, part_hbm, idx_v, hist_v):
    c = lax.axis_index("c")
    s = lax.axis_index("s")
    wid = c * 16 + s

    @pl.loop(0, NP // 16)
    def _(r):
        hist_v[pl.ds(r * 16, 16)] = jnp.zeros((16,), jnp.float32)

    @pl.loop(0, (NCHUNK // 32) // IG)
    def _(g):
        pltpu.sync_copy(dstw_hbm.at[pl.ds(wid * (NCHUNK // 32) + g * IG, IG)],
                        idx_v)
        for i in range(IG):
            for v in range(CH // 16):
                d16 = idx_v[i, pl.ds(v * 16, 16)]
                plsc.addupdate_scatter(hist_v, [d16],
                                       jnp.full((16,), 1.0, jnp.float32))

    pltpu.sync_copy(hist_v, part_hbm.at[pl.ds(wid * NP, NP)])


@functools.partial(
    pl.kernel,
    out_type=jax.ShapeDtypeStruct((NP // 2, 128), jnp.float32),
    mesh=_MESH,
    scratch_types=[
        pltpu.VMEM((RTILE,), jnp.float32),
        pltpu.VMEM((RTILE,), jnp.float32),
        pltpu.VMEM((16, 128), jnp.float32),
    ],
    compiler_params=_CP,
)
def _sc_degree_reduce(part_hbm, out_hbm, hbuf_v, red_v, outb_v):
    c = lax.axis_index("c")
    s = lax.axis_index("s")
    wid = c * 16 + s

    @pl.loop(0, RTILE // 16)
    def _(r):
        red_v[pl.ds(r * 16, 16)] = jnp.zeros((16,), jnp.float32)

    for w in range(32):
        pltpu.sync_copy(part_hbm.at[pl.ds(w * NP + wid * RTILE, RTILE)],
                        hbuf_v)

        @pl.loop(0, RTILE // 16)
        def _(r):
            red_v[pl.ds(r * 16, 16)] = (red_v[pl.ds(r * 16, 16)]
                                        + hbuf_v[pl.ds(r * 16, 16)])

    @pl.loop(0, RTILE // 32)
    def _(blk):
        @pl.loop(0, 16)
        def _(a):
            for b in range(2):
                ii = jnp.full((16,), blk * 32 + a * 2 + b, jnp.int32)
                vv = plsc.load_gather(red_v, [ii])
                for t in range(4):
                    outb_v[a, pl.ds(b * 64 + t * 16, 16)] = vv

        pltpu.sync_copy(outb_v,
                        out_hbm.at[pl.ds(wid * (RTILE // 2) + blk * 16, 16)])


def _sc_degree(dstw):
    return _sc_degree_reduce(_sc_degree_part(dstw))


def _make_sc_agg(F):
    CPT = F // 32

    @functools.partial(
        pl.kernel,
        out_type=jax.ShapeDtypeStruct((F * NP,), jnp.float32),
        mesh=_MESH,
        scratch_types=[
            pltpu.VMEM((IGA, CH), jnp.int32),
            pltpu.VMEM((IGA, CH), jnp.int32),
            pltpu.VMEM((CPT * NP,), jnp.float32),
            pltpu.VMEM((CPT * NP,), jnp.float32),
        ],
        compiler_params=_CP,
    )
    def _sc_agg(xst_hbm, srcw_hbm, dstw_hbm, out_hbm,
                sidx_v, didx_v, xs_v, acc_v):
        c = lax.axis_index("c")
        s = lax.axis_index("s")
        wid = c * 16 + s

        for t in range(CPT):
            pltpu.sync_copy(xst_hbm.at[pl.ds((wid * CPT + t) * NP, NP)],
                            xs_v.at[pl.ds(t * NP, NP)])

        @pl.loop(0, (CPT * NP) // 16)
        def _(r):
            acc_v[pl.ds(r * 16, 16)] = jnp.zeros((16,), jnp.float32)

        @pl.loop(0, NCHUNK // IGA)
        def _(g):
            pltpu.sync_copy(srcw_hbm.at[pl.ds(g * IGA, IGA)], sidx_v)
            pltpu.sync_copy(dstw_hbm.at[pl.ds(g * IGA, IGA)], didx_v)
            for i in range(IGA):
                for v in range(CH // 16):
                    s16 = sidx_v[i, pl.ds(v * 16, 16)]
                    d16 = didx_v[i, pl.ds(v * 16, 16)]
                    for t in range(CPT):
                        off = t * NP
                        vals = plsc.load_gather(xs_v, [s16 + off])
                        plsc.addupdate_scatter(acc_v, [d16 + off], vals)

        for t in range(CPT):
            pltpu.sync_copy(acc_v.at[pl.ds(t * NP, NP)],
                            out_hbm.at[pl.ds((wid * CPT + t) * NP, NP)])

    return _sc_agg


_sc_agg1 = _make_sc_agg(32)
_sc_agg2 = _make_sc_agg(64)


_BR = 1024


def _dis_from(deg_ref):
    return lax.rsqrt(deg_ref[...] + 1.0)


def _mm1_body(x_ref, w_ref, o_ref):
    o_ref[...] = jnp.dot(x_ref[...], w_ref[...],
                         preferred_element_type=jnp.float32)


def _tc_mm1(x, w):
    return pl.pallas_call(
        _mm1_body,
        grid=(NP // _BR,),
        in_specs=[
            pl.BlockSpec((_BR, 256), lambda i: (i, 0)),
            pl.BlockSpec((256, 32), lambda i: (0, 0)),
        ],
        out_specs=pl.BlockSpec((_BR, 32), lambda i: (i, 0)),
        out_shape=jax.ShapeDtypeStruct((NP, 32), jnp.float32),
    )(x, w)


def _scale_body(xw_ref, deg_ref, o_ref):
    dis = _dis_from(deg_ref)
    o_ref[...] = xw_ref[...] * dis[:, :32]


def _tc_scale(xw, deg):
    return pl.pallas_call(
        _scale_body,
        grid=(NP // _BR,),
        in_specs=[
            pl.BlockSpec((_BR, 32), lambda i: (i, 0)),
            pl.BlockSpec((_BR, 64), lambda i: (i, 0)),
        ],
        out_specs=pl.BlockSpec((_BR, 32), lambda i: (i, 0)),
        out_shape=jax.ShapeDtypeStruct((NP, 32), jnp.float32),
    )(xw, deg)


def _mid_body(p_ref, xw1_ref, deg_ref, b1_ref, w2_ref, xw2_ref, xs2_ref):
    dis = _dis_from(deg_ref)
    d32 = dis[:, :32]
    h1 = d32 * p_ref[...] + d32 * d32 * xw1_ref[...]
    h1 = jnp.maximum(h1 + b1_ref[0][None, :], 0.0)
    xw2 = jnp.dot(h1, w2_ref[...], preferred_element_type=jnp.float32)
    xw2_ref[...] = xw2
    xs2_ref[...] = xw2 * dis


def _tc_mid(p, xw1, deg, b1, w2):
    return pl.pallas_call(
        _mid_body,
        grid=(NP // _BR,),
        in_specs=[
            pl.BlockSpec((_BR, 32), lambda i: (i, 0)),
            pl.BlockSpec((_BR, 32), lambda i: (i, 0)),
            pl.BlockSpec((_BR, 64), lambda i: (i, 0)),
            pl.BlockSpec((1, 32), lambda i: (0, 0)),
            pl.BlockSpec((32, 64), lambda i: (0, 0)),
        ],
        out_specs=[
            pl.BlockSpec((_BR, 64), lambda i: (i, 0)),
            pl.BlockSpec((_BR, 64), lambda i: (i, 0)),
        ],
        out_shape=[
            jax.ShapeDtypeStruct((NP, 64), jnp.float32),
            jax.ShapeDtypeStruct((NP, 64), jnp.float32),
        ],
    )(p, xw1, deg, b1, w2)


def _out_body(q_ref, xw2_ref, deg_ref, b2_ref, o_ref):
    dis = _dis_from(deg_ref)
    logits = dis * q_ref[...] + dis * dis * xw2_ref[...]
    logits = jnp.maximum(logits + b2_ref[0][None, :], 0.0)
    m = jnp.max(logits, axis=1, keepdims=True)
    e = jnp.exp(logits - m)
    o_ref[...] = e / jnp.sum(e, axis=1, keepdims=True)


def _tc_out(q, xw2, deg, b2):
    return pl.pallas_call(
        _out_body,
        grid=(NP // _BR,),
        in_specs=[
            pl.BlockSpec((_BR, 64), lambda i: (i, 0)),
            pl.BlockSpec((_BR, 64), lambda i: (i, 0)),
            pl.BlockSpec((_BR, 64), lambda i: (i, 0)),
            pl.BlockSpec((1, 64), lambda i: (0, 0)),
        ],
        out_specs=pl.BlockSpec((_BR, 64), lambda i: (i, 0)),
        out_shape=jax.ShapeDtypeStruct((NP, 64), jnp.float32),
    )(q, xw2, deg, b2)


@jax.jit
def kernel(x, edge_index, W1, b1, W2, b2):
    ei = edge_index.astype(jnp.int32)
    src = jnp.concatenate([ei[0], jnp.zeros((EP - E,), jnp.int32)])
    dst = jnp.concatenate([ei[1], jnp.full((EP - E,), NP - 1, jnp.int32)])
    srcw = src.reshape(NCHUNK, CH)
    dstw = dst.reshape(NCHUNK, CH)
    xp = jnp.concatenate([x, jnp.zeros((NP - N, x.shape[1]), x.dtype)])

    deg = _sc_degree(dstw)
    degr = deg.reshape(NP, 64)
    xw1 = _tc_mm1(xp, W1)
    xs1 = _tc_scale(xw1, degr)
    p = _sc_agg1(xs1.T.reshape(-1), srcw, dstw)
    xw2, xs2 = _tc_mid(p.reshape(32, NP).T, xw1, degr,
                       b1.reshape(1, 32), W2)
    q = _sc_agg2(xs2.T.reshape(-1), srcw, dstw)
    probs = _tc_out(q.reshape(64, NP).T, xw2, degr, b2.reshape(1, 64))
    return probs[:N]
